# Initial kernel scaffold; baseline (speedup 1.0000x reference)
import jax
import jax.numpy as jnp
from jax.experimental import pallas as pl


def kernel(hidden_states, W1, b1, W2, b2):
    raise NotImplementedError("write your pallas kernel here")


# speedup vs baseline: 2.2036x; 2.2036x over previous
import jax
import jax.numpy as jnp
from jax.experimental import pallas as pl
from jax.experimental.pallas import tpu as pltpu

HIDDEN = 4096
FF = 2048
E = 64
TOP_K = 8
THRESHOLD = 0.2
TBLK = 128
_BASE_BITS = 1045220557


def _router_kernel(x_ref, w1_hbm, b1_ref, w2_ref, b2_ref,
                   logits_ref, mask_ref, normw_ref, loss_ref,
                   w1_ref, acc_mask, acc_prob, w1_sem):
    i = pl.program_id(0)
    n = pl.num_programs(0)

    @pl.when(i == 0)
    def _load_w1():
        pltpu.make_async_copy(w1_hbm, w1_ref, w1_sem).start()
        pltpu.make_async_copy(w1_hbm, w1_ref, w1_sem).wait()

    x = x_ref[...]
    h = jax.lax.dot_general(x, w1_ref[...], (((1,), (0,)), ((), ())),
                            preferred_element_type=jnp.float32,
                            precision=jax.lax.Precision.DEFAULT)
    h = jnp.maximum(h + b1_ref[...], 0.0)
    logits = jax.lax.dot_general(h, w2_ref[...], (((1,), (0,)), ((), ())),
                                 preferred_element_type=jnp.float32,
                                 precision=jax.lax.Precision.DEFAULT)
    logits = logits + b2_ref[...]
    logits_ref[...] = logits

    gates = jax.nn.sigmoid(logits)
    pos = gates > THRESHOLD

    bits = jax.lax.bitcast_convert_type(gates, jnp.int32)
    eidx = jax.lax.broadcasted_iota(jnp.int32, gates.shape, 1)
    key = ((bits - _BASE_BITS) << 6) | ((E - 1) - eidx)
    cur = jnp.where(pos, key, -1)
    sel = jnp.zeros(gates.shape, dtype=jnp.bool_)
    for _ in range(TOP_K):
        m = jnp.max(cur, axis=1, keepdims=True)
        hit = cur == m
        sel = sel | hit
        cur = jnp.where(hit, -1, cur)
    maskf = jnp.where(sel & pos, 1.0, 0.0)
    mask_ref[...] = maskf

    w = gates * maskf
    normw_ref[...] = w / (jnp.sum(w, axis=1, keepdims=True) + 1e-6)

    mx = jnp.max(logits, axis=1, keepdims=True)
    ex = jnp.exp(logits - mx)
    probs = ex / jnp.sum(ex, axis=1, keepdims=True)

    mm = jnp.sum(maskf, axis=0, keepdims=True)
    pm = jnp.sum(probs, axis=0, keepdims=True)

    @pl.when(i == 0)
    def _init():
        acc_mask[...] = mm
        acc_prob[...] = pm

    @pl.when(i > 0)
    def _acc():
        acc_mask[...] += mm
        acc_prob[...] += pm

    @pl.when(i == n - 1)
    def _fin():
        t = jnp.float32(n * TBLK)
        s = jnp.sum(acc_mask[...] * acc_prob[...], axis=1, keepdims=True)
        loss_ref[...] = s * jnp.float32(E) / (t * t)


def kernel(hidden_states, W1, b1, W2, b2):
    B, S, H = hidden_states.shape
    T = B * S
    x = hidden_states.reshape(T, H)
    w1t = W1.T
    w2t = W2.T
    grid = (T // TBLK,)

    out_shape = [
        jax.ShapeDtypeStruct((T, E), jnp.float32),
        jax.ShapeDtypeStruct((T, E), jnp.float32),
        jax.ShapeDtypeStruct((T, E), jnp.float32),
        jax.ShapeDtypeStruct((1, 1), jnp.float32),
    ]
    in_specs = [
        pl.BlockSpec((TBLK, H), lambda i: (i, 0)),
        pl.BlockSpec(memory_space=pltpu.MemorySpace.HBM),
        pl.BlockSpec((1, FF), lambda i: (0, 0)),
        pl.BlockSpec((FF, E), lambda i: (0, 0)),
        pl.BlockSpec((1, E), lambda i: (0, 0)),
    ]
    out_specs = [
        pl.BlockSpec((TBLK, E), lambda i: (i, 0)),
        pl.BlockSpec((TBLK, E), lambda i: (i, 0)),
        pl.BlockSpec((TBLK, E), lambda i: (i, 0)),
        pl.BlockSpec((1, 1), lambda i: (0, 0)),
    ]
    logits, maskf, normw, loss = pl.pallas_call(
        _router_kernel,
        grid=grid,
        in_specs=in_specs,
        out_specs=out_specs,
        out_shape=out_shape,
        scratch_shapes=[pltpu.VMEM((H, FF), jnp.float32),
                        pltpu.VMEM((1, E), jnp.float32),
                        pltpu.VMEM((1, E), jnp.float32),
                        pltpu.SemaphoreType.DMA],
        compiler_params=pltpu.CompilerParams(
            dimension_semantics=("arbitrary",)),
    )(x, w1t, b1.reshape(1, FF), w2t, b2.reshape(1, E))

    dispatch_mask = maskf.astype(bool).reshape(B, S, E)
    normalized_weights = normw.reshape(B, S, E)
    router_logits = logits.reshape(B, S, E)
    return dispatch_mask, normalized_weights, loss[0, 0], router_logits


# speedup vs baseline: 2.7639x; 1.2543x over previous
import jax
import jax.numpy as jnp
from jax.experimental import pallas as pl
from jax.experimental.pallas import tpu as pltpu

HIDDEN = 4096
FF = 2048
E = 64
TOP_K = 8
THRESHOLD = 0.2
TBLK = 256
_BASE_BITS = 1045220557


def _router_kernel(x_ref, w1_hbm, b1_ref, w2_ref, b2_ref,
                   logits_ref, mask_ref, normw_ref, loss_ref,
                   w1_ref, acc_mask, acc_prob, w1_sem):
    i = pl.program_id(0)
    n = pl.num_programs(0)

    @pl.when(i == 0)
    def _load_w1():
        pltpu.make_async_copy(w1_hbm, w1_ref, w1_sem).start()
        pltpu.make_async_copy(w1_hbm, w1_ref, w1_sem).wait()

    x = x_ref[...].astype(jnp.bfloat16)
    h = jax.lax.dot_general(x, w1_ref[...], (((1,), (0,)), ((), ())),
                            preferred_element_type=jnp.float32,
                            precision=jax.lax.Precision.DEFAULT)
    h = jnp.maximum(h + b1_ref[...], 0.0)
    logits = jax.lax.dot_general(h, w2_ref[...], (((1,), (0,)), ((), ())),
                                 preferred_element_type=jnp.float32,
                                 precision=jax.lax.Precision.DEFAULT)
    logits = logits + b2_ref[...]
    logits_ref[...] = logits

    gates = jax.nn.sigmoid(logits)
    pos = gates > THRESHOLD

    bits = jax.lax.bitcast_convert_type(gates, jnp.int32)
    eidx = jax.lax.broadcasted_iota(jnp.int32, gates.shape, 1)
    key = ((bits - _BASE_BITS) << 6) | ((E - 1) - eidx)
    cur = jnp.where(pos, key, -1)
    sel = jnp.zeros(gates.shape, dtype=jnp.bool_)
    for _ in range(TOP_K):
        m = jnp.max(cur, axis=1, keepdims=True)
        hit = cur == m
        sel = sel | hit
        cur = jnp.where(hit, -1, cur)
    maskf = jnp.where(sel & pos, 1.0, 0.0)
    mask_ref[...] = maskf

    w = gates * maskf
    normw_ref[...] = w / (jnp.sum(w, axis=1, keepdims=True) + 1e-6)

    mx = jnp.max(logits, axis=1, keepdims=True)
    ex = jnp.exp(logits - mx)
    probs = ex / jnp.sum(ex, axis=1, keepdims=True)

    mm = jnp.sum(maskf, axis=0, keepdims=True)
    pm = jnp.sum(probs, axis=0, keepdims=True)

    @pl.when(i == 0)
    def _init():
        acc_mask[...] = mm
        acc_prob[...] = pm

    @pl.when(i > 0)
    def _acc():
        acc_mask[...] += mm
        acc_prob[...] += pm

    @pl.when(i == n - 1)
    def _fin():
        t = jnp.float32(n * TBLK)
        s = jnp.sum(acc_mask[...] * acc_prob[...], axis=1, keepdims=True)
        loss_ref[...] = s * jnp.float32(E) / (t * t)


def kernel(hidden_states, W1, b1, W2, b2):
    B, S, H = hidden_states.shape
    T = B * S
    x = hidden_states.reshape(T, H)
    w1t = W1.T.astype(jnp.bfloat16)
    w2t = W2.T
    grid = (T // TBLK,)

    out_shape = [
        jax.ShapeDtypeStruct((T, E), jnp.float32),
        jax.ShapeDtypeStruct((T, E), jnp.float32),
        jax.ShapeDtypeStruct((T, E), jnp.float32),
        jax.ShapeDtypeStruct((1, 1), jnp.float32),
    ]
    in_specs = [
        pl.BlockSpec((TBLK, H), lambda i: (i, 0)),
        pl.BlockSpec(memory_space=pltpu.MemorySpace.HBM),
        pl.BlockSpec((1, FF), lambda i: (0, 0)),
        pl.BlockSpec((FF, E), lambda i: (0, 0)),
        pl.BlockSpec((1, E), lambda i: (0, 0)),
    ]
    out_specs = [
        pl.BlockSpec((TBLK, E), lambda i: (i, 0)),
        pl.BlockSpec((TBLK, E), lambda i: (i, 0)),
        pl.BlockSpec((TBLK, E), lambda i: (i, 0)),
        pl.BlockSpec((1, 1), lambda i: (0, 0)),
    ]
    logits, maskf, normw, loss = pl.pallas_call(
        _router_kernel,
        grid=grid,
        in_specs=in_specs,
        out_specs=out_specs,
        out_shape=out_shape,
        scratch_shapes=[pltpu.VMEM((H, FF), jnp.bfloat16),
                        pltpu.VMEM((1, E), jnp.float32),
                        pltpu.VMEM((1, E), jnp.float32),
                        pltpu.SemaphoreType.DMA],
        compiler_params=pltpu.CompilerParams(
            dimension_semantics=("arbitrary",)),
    )(x, w1t, b1.reshape(1, FF), w2t, b2.reshape(1, E))

    dispatch_mask = maskf.astype(bool).reshape(B, S, E)
    normalized_weights = normw.reshape(B, S, E)
    router_logits = logits.reshape(B, S, E)
    return dispatch_mask, normalized_weights, loss[0, 0], router_logits


# speedup vs baseline: 3.0887x; 1.1175x over previous
import jax
import jax.numpy as jnp
from jax.experimental import pallas as pl
from jax.experimental.pallas import tpu as pltpu

HIDDEN = 4096
FF = 2048
E = 64
TOP_K = 8
THRESHOLD = 0.2
TBLK = 512
_BASE_BITS = 1045220557


def _router_kernel(x_ref, w1_hbm, b1_ref, w2_ref, b2_ref,
                   logits_ref, mask_ref, normw_ref, loss_ref,
                   w1_ref, acc_mask, acc_prob, w1_sem):
    i = pl.program_id(0)
    n = pl.num_programs(0)

    @pl.when(i == 0)
    def _load_w1():
        pltpu.make_async_copy(w1_hbm, w1_ref, w1_sem).start()
        pltpu.make_async_copy(w1_hbm, w1_ref, w1_sem).wait()

    x = x_ref[...].astype(jnp.bfloat16)
    h = jax.lax.dot_general(x, w1_ref[...], (((1,), (0,)), ((), ())),
                            preferred_element_type=jnp.float32,
                            precision=jax.lax.Precision.DEFAULT)
    h = jnp.maximum(h + b1_ref[...], 0.0)
    logits = jax.lax.dot_general(h, w2_ref[...], (((1,), (0,)), ((), ())),
                                 preferred_element_type=jnp.float32,
                                 precision=jax.lax.Precision.DEFAULT)
    logits = logits + b2_ref[...]
    logits_ref[...] = logits

    gates = jax.nn.sigmoid(logits)
    pos = gates > THRESHOLD

    bits = jax.lax.bitcast_convert_type(gates, jnp.int32)
    eidx = jax.lax.broadcasted_iota(jnp.int32, gates.shape, 1)
    key = ((bits - _BASE_BITS) << 6) | ((E - 1) - eidx)
    cur = jnp.where(pos, key, -1)
    sel = jnp.zeros(gates.shape, dtype=jnp.bool_)
    for _ in range(TOP_K):
        m = jnp.max(cur, axis=1, keepdims=True)
        hit = cur == m
        sel = sel | hit
        cur = jnp.where(hit, -1, cur)
    maskf = jnp.where(sel & pos, 1.0, 0.0)
    mask_ref[...] = maskf

    w = gates * maskf
    normw_ref[...] = w / (jnp.sum(w, axis=1, keepdims=True) + 1e-6)

    mx = jnp.max(logits, axis=1, keepdims=True)
    ex = jnp.exp(logits - mx)
    probs = ex / jnp.sum(ex, axis=1, keepdims=True)

    mm = jnp.sum(maskf, axis=0, keepdims=True)
    pm = jnp.sum(probs, axis=0, keepdims=True)

    @pl.when(i == 0)
    def _init():
        acc_mask[...] = mm
        acc_prob[...] = pm

    @pl.when(i > 0)
    def _acc():
        acc_mask[...] += mm
        acc_prob[...] += pm

    @pl.when(i == n - 1)
    def _fin():
        t = jnp.float32(n * TBLK)
        s = jnp.sum(acc_mask[...] * acc_prob[...], axis=1, keepdims=True)
        loss_ref[...] = s * jnp.float32(E) / (t * t)


def kernel(hidden_states, W1, b1, W2, b2):
    B, S, H = hidden_states.shape
    T = B * S
    x = hidden_states.reshape(T, H)
    w1t = W1.T.astype(jnp.bfloat16)
    w2t = W2.T
    grid = (T // TBLK,)

    out_shape = [
        jax.ShapeDtypeStruct((T, E), jnp.float32),
        jax.ShapeDtypeStruct((T, E), jnp.float32),
        jax.ShapeDtypeStruct((T, E), jnp.float32),
        jax.ShapeDtypeStruct((1, 1), jnp.float32),
    ]
    in_specs = [
        pl.BlockSpec((TBLK, H), lambda i: (i, 0)),
        pl.BlockSpec(memory_space=pltpu.MemorySpace.HBM),
        pl.BlockSpec((1, FF), lambda i: (0, 0)),
        pl.BlockSpec((FF, E), lambda i: (0, 0)),
        pl.BlockSpec((1, E), lambda i: (0, 0)),
    ]
    out_specs = [
        pl.BlockSpec((TBLK, E), lambda i: (i, 0)),
        pl.BlockSpec((TBLK, E), lambda i: (i, 0)),
        pl.BlockSpec((TBLK, E), lambda i: (i, 0)),
        pl.BlockSpec((1, 1), lambda i: (0, 0)),
    ]
    logits, maskf, normw, loss = pl.pallas_call(
        _router_kernel,
        grid=grid,
        in_specs=in_specs,
        out_specs=out_specs,
        out_shape=out_shape,
        scratch_shapes=[pltpu.VMEM((H, FF), jnp.bfloat16),
                        pltpu.VMEM((1, E), jnp.float32),
                        pltpu.VMEM((1, E), jnp.float32),
                        pltpu.SemaphoreType.DMA],
        compiler_params=pltpu.CompilerParams(
            dimension_semantics=("arbitrary",)),
    )(x, w1t, b1.reshape(1, FF), w2t, b2.reshape(1, E))

    dispatch_mask = maskf.astype(bool).reshape(B, S, E)
    normalized_weights = normw.reshape(B, S, E)
    router_logits = logits.reshape(B, S, E)
    return dispatch_mask, normalized_weights, loss[0, 0], router_logits
